# Initial kernel scaffold; baseline (speedup 1.0000x reference)
import jax
import jax.numpy as jnp
from jax.experimental import pallas as pl


def kernel(x):
    raise NotImplementedError("write your pallas kernel here")


# speedup vs baseline: 3.7147x; 3.7147x over previous
import functools
import numpy as np
import jax
import jax.numpy as jnp
from jax import lax
from jax.experimental import pallas as pl
from jax.experimental.pallas import tpu as pltpu
from jax.experimental.pallas import tpu_sc as plsc

MIN_C = 1
MAX_C = 16
TOP_P = 10

B, C, H, W = 4, 96, 384, 384
NPIX = H * W
NUM_TOP = int(NPIX * TOP_P / 100)

_ranks = 1.0 - np.arange(NUM_TOP, dtype=np.float32) / np.float32(NUM_TOP)
_cv = (MIN_C + np.round(_ranks * _ranks * np.float32(MAX_C - MIN_C))).astype(np.int32)
_BK = [int((_cv >= k + 1).sum()) for k in range(1, 16)]


C_BLK = 8


def _upfilter(a, axis, par):
    n = a.shape[axis]

    def sl(lo, hi):
        idx = [slice(None)] * a.ndim
        idx[axis] = slice(lo, hi)
        return a[tuple(idx)]

    nxt = jnp.concatenate([sl(1, n), sl(n - 1, n)], axis=axis)
    prv = jnp.concatenate([sl(0, 1), sl(0, n - 1)], axis=axis)
    E = a + jnp.where(par, nxt, prv)

    def slE(lo, hi):
        idx = [slice(None)] * a.ndim
        idx[axis] = slice(lo, hi)
        return E[tuple(idx)]

    Ep = jnp.concatenate([slE(0, 2), slE(0, n - 2)], axis=axis)
    En = jnp.concatenate([slE(2, n), slE(n - 2, n)], axis=axis)
    Es = jnp.where(par, Ep, En)
    return 0.375 * E + 0.125 * Es


def _df_body(x_ref, out_ref):
    xb = x_ref[0]
    par_h = (lax.broadcasted_iota(jnp.int32, (1, H, 1), 1) % 2) == 0
    par_w = (lax.broadcasted_iota(jnp.int32, (1, 1, W), 2) % 2) == 0
    u = _upfilter(_upfilter(xb, 1, par_h), 2, par_w)
    d = jnp.sum(jnp.abs(xb - u), axis=0)

    @pl.when(pl.program_id(1) == 0)
    def _init():
        out_ref[0] = d

    @pl.when(pl.program_id(1) != 0)
    def _acc():
        out_ref[0] += d


def _df_stage(x):
    return pl.pallas_call(
        _df_body,
        grid=(B, C // C_BLK),
        in_specs=[pl.BlockSpec((1, C_BLK, H, W), lambda b, c: (b, c, 0, 0))],
        out_specs=pl.BlockSpec((1, H, W), lambda b, c: (b, 0, 0)),
        out_shape=jax.ShapeDtypeStruct((B, H, W), jnp.float32),
    )(x)


def _conn_body(thr_ref, df_ref, out_ref):
    d = df_ref[0]
    acc = jnp.ones(d.shape, jnp.int32)
    for k in range(15):
        acc += jnp.where(d > thr_ref[0, 0, k], 1, 0).astype(jnp.int32)
    out_ref[0] = acc


def _conn_stage(df, thr_f32):
    return pl.pallas_call(
        _conn_body,
        grid=(B,),
        in_specs=[
            pl.BlockSpec(
                (1, 1, 16), lambda b: (b, 0, 0), memory_space=pltpu.SMEM
            ),
            pl.BlockSpec((1, H, W), lambda b: (b, 0, 0)),
        ],
        out_specs=pl.BlockSpec((1, H, W), lambda b: (b, 0, 0)),
        out_shape=jax.ShapeDtypeStruct((B, H, W), jnp.int32),
    )(thr_f32.reshape(B, 1, 16), df)


def kernel(x):
    df = _df_stage(x)
    keys = lax.bitcast_convert_type(df, jnp.int32).reshape(B * NPIX)
    thr, _ = _select_stage(keys)
    thr_f = lax.bitcast_convert_type(thr.reshape(B, 16), jnp.float32)
    conn = _conn_stage(df, thr_f)
    return (df[:, None], conn[:, None])


def _select_stage(keys):
    mesh = plsc.VectorSubcoreMesh(core_axis_name="c", subcore_axis_name="s")
    CHUNK = NPIX // 8

    @functools.partial(
        pl.kernel,
        mesh=mesh,
        out_type=(jax.ShapeDtypeStruct((B * 16,), jnp.int32),
                  jax.ShapeDtypeStruct((B * 8 * 32, 16), jnp.int32)),
        compiler_params=pltpu.CompilerParams(needs_layout_passes=False),
        scratch_types=[
            pltpu.VMEM((CHUNK,), jnp.int32),
            pltpu.VMEM((32768,), jnp.int32),
            pltpu.VMEM((128, 16), jnp.int32),
            pltpu.VMEM((128, 16), jnp.int32),
            pltpu.VMEM((2048,), jnp.int32),
            pltpu.VMEM((2048,), jnp.int32),
            pltpu.VMEM((2048,), jnp.int32),
            pltpu.VMEM((2048,), jnp.int32),
            pltpu.VMEM((256, 16), jnp.int32),
            pltpu.VMEM((16,), jnp.int32),
            pltpu.VMEM((16,), jnp.int32),
        ],
    )
    def sel(keys_hbm, tgt_hbm, out_hbm, ex_hbm, keys_v, hist, merged, mg, S,
            lut2, lut3, lut4, grids, tstage, tgt_v):
        cid = lax.axis_index("c")
        sid = lax.axis_index("s")
        bl = sid // 8
        batch = cid * 2 + bl
        chunk = sid % 8
        leader = chunk == 0

        lane = lax.iota(jnp.int32, 16)
        zvec = lane * 0
        ones = zvec + 1

        pltpu.sync_copy(tgt_hbm, tgt_v)
        targets0 = tgt_v[...]

        pltpu.sync_copy(
            keys_hbm.at[pl.ds(batch * NPIX + chunk * CHUNK, CHUNK)], keys_v)

        def zero_hist():
            def bd(i, _):
                hist[pl.ds(i * 16, 16)] = zvec
                return 0
            lax.fori_loop(0, 2048, bd, 0)

        def lane_reduce_p1():
            def bd(i, _):
                acc = zvec
                for l in range(16):
                    acc = acc + hist[pl.ds(l * 2048 + i * 16, 16)]
                merged[i, :] = acc
                return 0
            lax.fori_loop(0, 128, bd, 0)

        def lane_reduce_p2():
            def bd(i, _):
                j = i // 8
                rc = i % 8
                acc = zvec
                for l in range(16):
                    acc = acc + hist[pl.ds(j * 2048 + l * 128 + rc * 16, 16)]
                merged[j * 8 + rc, :] = acc
                return 0
            lax.fori_loop(0, 128, bd, 0)

        def publish_fetch(p):
            for r in range(4):
                pltpu.sync_copy(
                    merged.at[pl.ds(r * 32, 32)],
                    ex_hbm.at[pl.ds((batch * 8 + chunk) * 32, 32)])
                plsc.subcore_barrier()
                pltpu.sync_copy(ex_hbm.at[pl.ds(batch * 8 * 32, 256)], grids)
                def bd(i, _):
                    acc = zvec
                    for q in range(8):
                        acc = acc + grids[q * 32 + i, :]
                    mg[r * 32 + i, :] = acc
                    return 0
                lax.fori_loop(0, 32, bd, 0)
                plsc.subcore_barrier()

        def suffix_scan_chain(lo_chunk, n_chunks):
            def bd(i, carry):
                c = lo_chunk + n_chunks - 1 - i
                v = mg[c, :]
                cs = lax.cumsum(v, axis=0)
                tot = jnp.sum(v, axis=0)
                S[pl.ds(c * 16, 16)] = carry + tot - cs
                return carry + tot
            return lax.fori_loop(0, n_chunks, bd, jnp.int32(0))

        zero_hist()

        def sweep1(i, _):
            k = keys_v[pl.ds(i * 16, 16)]
            b = lax.shift_right_logical(k, 21)
            plsc.addupdate_scatter(hist, [lane * 2048 + b], ones)
            return 0
        lax.fori_loop(0, CHUNK // 16, sweep1, 0)
        lane_reduce_p1()
        publish_fetch(0)
        suffix_scan_chain(0, 128)

        def search_full(T):
            def bd(b, acc):
                sb = plsc.load_gather(S, [zvec + b])
                return acc + (sb > T).astype(jnp.int32)
            return lax.fori_loop(0, 2048, bd, lane * 0)

        b1 = search_full(targets0)
        Sg = plsc.load_gather(S, [b1])
        T1 = targets0 - Sg

        def build_lut(lut_ref, keyvec):
            def bd(i, _):
                lut_ref[pl.ds(i * 16, 16)] = zvec + 30720
                return 0
            lax.fori_loop(0, 128, bd, 0)
            plsc.store_scatter(lut_ref, [keyvec], lane * 2048,
                               mask=lane < 15)
            return plsc.load_gather(lut_ref, [keyvec])

        slot2 = build_lut(lut2, b1)

        def slot_pass(p, shift, lut_prev_chain, T_in, slot_in):
            zero_hist()

            def sweep(i, _):
                k = keys_v[pl.ds(i * 16, 16)]
                bits = lax.shift_right_logical(k, shift) & 127
                base = lut_prev_chain(k)
                idx = base + lane * 128 + bits
                plsc.addupdate_scatter(hist, [idx], ones)
                return 0
            lax.fori_loop(0, CHUNK // 16, sweep, 0)
            lane_reduce_p2()
            publish_fetch(p)

            def bd(j, _):
                suffix_scan_chain(j * 8, 8)
                return 0
            lax.fori_loop(0, 16, bd, 0)

            slot_row = lax.shift_right_logical(slot_in, 11) * 128

            def sbd(r, acc):
                sg = plsc.load_gather(S, [slot_row + r])
                return acc + (sg > T_in).astype(jnp.int32)
            bp = lax.fori_loop(0, 128, sbd, lane * 0)
            Sg2 = plsc.load_gather(S, [slot_row + bp])
            T_out = T_in - Sg2
            return bp, T_out

        def chain2(k):
            b = lax.shift_right_logical(k, 21)
            return plsc.load_gather(lut2, [b])

        b2, T2 = slot_pass(1, 14, chain2, T1, slot2)

        key3 = lax.shift_right_logical(slot2, 11) * 128 + b2
        slot3 = build_lut(lut3, key3)

        def chain3(k):
            s2 = chain2(k)
            bits2 = lax.shift_right_logical(k, 14) & 127
            return plsc.load_gather(
                lut3, [lax.shift_right_logical(s2, 11) * 128 + bits2])

        b3, T3 = slot_pass(2, 7, chain3, T2, slot3)

        key4 = lax.shift_right_logical(slot3, 11) * 128 + b3
        slot4 = build_lut(lut4, key4)

        def chain4(k):
            s3 = chain3(k)
            bits3 = lax.shift_right_logical(k, 7) & 127
            return plsc.load_gather(
                lut4, [lax.shift_right_logical(s3, 11) * 128 + bits3])

        b4, _T4 = slot_pass(3, 0, chain4, T3, slot4)

        tvec = (b1 * (1 << 21)) + (b2 * (1 << 14)) + (b3 * (1 << 7)) + b4

        @pl.when(leader)
        def _out():
            tstage[...] = tvec
            pltpu.sync_copy(tstage, out_hbm.at[pl.ds(batch * 16, 16)])

    return sel(keys, jnp.asarray(_BK + [0x7FFFFFFF], dtype=jnp.int32))


# speedup vs baseline: 4.0249x; 1.0835x over previous
import functools
import numpy as np
import jax
import jax.numpy as jnp
from jax import lax
from jax.experimental import pallas as pl
from jax.experimental.pallas import tpu as pltpu
from jax.experimental.pallas import tpu_sc as plsc

MIN_C = 1
MAX_C = 16
TOP_P = 10

B, C, H, W = 4, 96, 384, 384
NPIX = H * W
NUM_TOP = int(NPIX * TOP_P / 100)

_ranks = 1.0 - np.arange(NUM_TOP, dtype=np.float32) / np.float32(NUM_TOP)
_cv = (MIN_C + np.round(_ranks * _ranks * np.float32(MAX_C - MIN_C))).astype(np.int32)
_BK = [int((_cv >= k + 1).sum()) for k in range(1, 16)]


C_BLK = 8


def _upfilter(a, axis, par):
    n = a.shape[axis]

    def sl(lo, hi):
        idx = [slice(None)] * a.ndim
        idx[axis] = slice(lo, hi)
        return a[tuple(idx)]

    nxt = jnp.concatenate([sl(1, n), sl(n - 1, n)], axis=axis)
    prv = jnp.concatenate([sl(0, 1), sl(0, n - 1)], axis=axis)
    E = a + jnp.where(par, nxt, prv)

    def slE(lo, hi):
        idx = [slice(None)] * a.ndim
        idx[axis] = slice(lo, hi)
        return E[tuple(idx)]

    Ep = jnp.concatenate([slE(0, 2), slE(0, n - 2)], axis=axis)
    En = jnp.concatenate([slE(2, n), slE(n - 2, n)], axis=axis)
    Es = jnp.where(par, Ep, En)
    return 0.375 * E + 0.125 * Es


def _df_body(x_ref, out_ref):
    xb = x_ref[0]
    par_h = (lax.broadcasted_iota(jnp.int32, (1, H, 1), 1) % 2) == 0
    par_w = (lax.broadcasted_iota(jnp.int32, (1, 1, W), 2) % 2) == 0
    u = _upfilter(_upfilter(xb, 1, par_h), 2, par_w)
    d = jnp.sum(jnp.abs(xb - u), axis=0)

    @pl.when(pl.program_id(1) == 0)
    def _init():
        out_ref[0] = d

    @pl.when(pl.program_id(1) != 0)
    def _acc():
        out_ref[0] += d


def _df_stage(x):
    return pl.pallas_call(
        _df_body,
        grid=(B, C // C_BLK),
        in_specs=[pl.BlockSpec((1, C_BLK, H, W), lambda b, c: (b, c, 0, 0))],
        out_specs=pl.BlockSpec((1, H, W), lambda b, c: (b, 0, 0)),
        out_shape=jax.ShapeDtypeStruct((B, H, W), jnp.float32),
    )(x)


def _conn_body(thr_ref, df_ref, out_ref):
    d = df_ref[0]
    acc = jnp.ones(d.shape, jnp.int32)
    for k in range(15):
        acc += jnp.where(d > thr_ref[0, 0, k], 1, 0).astype(jnp.int32)
    out_ref[0] = acc


def _conn_stage(df, thr_f32):
    return pl.pallas_call(
        _conn_body,
        grid=(B,),
        in_specs=[
            pl.BlockSpec(
                (1, 1, 16), lambda b: (b, 0, 0), memory_space=pltpu.SMEM
            ),
            pl.BlockSpec((1, H, W), lambda b: (b, 0, 0)),
        ],
        out_specs=pl.BlockSpec((1, H, W), lambda b: (b, 0, 0)),
        out_shape=jax.ShapeDtypeStruct((B, H, W), jnp.int32),
    )(thr_f32.reshape(B, 1, 16), df)


def kernel(x):
    df = _df_stage(x)
    keys = lax.bitcast_convert_type(df, jnp.int32).reshape(B * NPIX)
    thr, _ = _select_stage(keys)
    thr_f = lax.bitcast_convert_type(thr.reshape(B, 16), jnp.float32)
    conn = _conn_stage(df, thr_f)
    return (df[:, None], conn[:, None])


def _select_stage(keys):
    mesh = plsc.VectorSubcoreMesh(core_axis_name="c", subcore_axis_name="s")
    CHUNK = NPIX // 8

    @functools.partial(
        pl.kernel,
        mesh=mesh,
        out_type=(jax.ShapeDtypeStruct((B * 16,), jnp.int32),
                  jax.ShapeDtypeStruct((B * 8 * 32, 16), jnp.int32)),
        compiler_params=pltpu.CompilerParams(needs_layout_passes=False),
        scratch_types=[
            pltpu.VMEM((CHUNK,), jnp.int32),
            pltpu.VMEM((32768,), jnp.int32),
            pltpu.VMEM((128, 16), jnp.int32),
            pltpu.VMEM((128, 16), jnp.int32),
            pltpu.VMEM((2048,), jnp.int32),
            pltpu.VMEM((2048,), jnp.int32),
            pltpu.VMEM((2048,), jnp.int32),
            pltpu.VMEM((2048,), jnp.int32),
            pltpu.VMEM((256, 16), jnp.int32),
            pltpu.VMEM((16,), jnp.int32),
            pltpu.VMEM((16,), jnp.int32),
        ],
    )
    def sel(keys_hbm, tgt_hbm, out_hbm, ex_hbm, keys_v, hist, merged, mg, S,
            lut2, lut3, lut4, grids, tstage, tgt_v):
        cid = lax.axis_index("c")
        sid = lax.axis_index("s")
        bl = sid // 8
        batch = cid * 2 + bl
        chunk = sid % 8
        leader = chunk == 0

        lane = lax.iota(jnp.int32, 16)
        zvec = lane * 0
        ones = zvec + 1

        pltpu.sync_copy(tgt_hbm, tgt_v)
        targets0 = tgt_v[...]

        pltpu.sync_copy(
            keys_hbm.at[pl.ds(batch * NPIX + chunk * CHUNK, CHUNK)], keys_v)

        def zero_hist():
            @plsc.parallel_loop(0, 2048, unroll=8)
            def _zh(i):
                hist[pl.ds(i * 16, 16)] = zvec

        def lane_reduce_p1():
            @plsc.parallel_loop(0, 128, unroll=2)
            def _lr1(i):
                acc = zvec
                for l in range(16):
                    acc = acc + hist[pl.ds(l * 2048 + i * 16, 16)]
                merged[i, :] = acc

        def lane_reduce_p2():
            @plsc.parallel_loop(0, 128, unroll=2)
            def _lr2(i):
                j = i // 8
                rc = i - j * 8
                acc = zvec
                for l in range(16):
                    acc = acc + hist[pl.ds(j * 2048 + l * 128 + rc * 16, 16)]
                merged[j * 8 + rc, :] = acc

        def publish_fetch(p):
            for r in range(4):
                pltpu.sync_copy(
                    merged.at[pl.ds(r * 32, 32)],
                    ex_hbm.at[pl.ds((batch * 8 + chunk) * 32, 32)])
                plsc.subcore_barrier()
                pltpu.sync_copy(ex_hbm.at[pl.ds(batch * 8 * 32, 256)], grids)
                @plsc.parallel_loop(0, 32, unroll=2)
                def _xs(i):
                    acc = zvec
                    for q in range(8):
                        acc = acc + grids[q * 32 + i, :]
                    mg[r * 32 + i, :] = acc
                plsc.subcore_barrier()

        def suffix_scan_chain(lo_chunk, n_chunks):
            def bd(i, carry):
                c = lo_chunk + n_chunks - 1 - i
                v = mg[c, :]
                cs = lax.cumsum(v, axis=0)
                tot = jnp.sum(v, axis=0)
                S[pl.ds(c * 16, 16)] = carry + tot - cs
                return carry + tot
            return lax.fori_loop(0, n_chunks, bd, jnp.int32(0))

        zero_hist()

        def sweep1(i, _):
            for u in range(4):
                k = keys_v[pl.ds(i * 64 + u * 16, 16)]
                b = lax.shift_right_logical(k, 21)
                plsc.addupdate_scatter(hist, [lane * 2048 + b], ones)
            return 0
        lax.fori_loop(0, CHUNK // 64, sweep1, 0)
        lane_reduce_p1()
        publish_fetch(0)
        suffix_scan_chain(0, 128)

        def search_full(T):
            @plsc.parallel_loop(0, 2048, unroll=8, carry=lane * 0)
            def acc_loop(b, acc):
                sb = plsc.load_gather(S, [zvec + b])
                return acc + (sb > T).astype(jnp.int32)
            return acc_loop

        b1 = search_full(targets0)
        Sg = plsc.load_gather(S, [b1])
        T1 = targets0 - Sg

        def build_lut(lut_ref, keyvec):
            @plsc.parallel_loop(0, 128, unroll=4)
            def _bl(i):
                lut_ref[pl.ds(i * 16, 16)] = zvec + 30720
            plsc.store_scatter(lut_ref, [keyvec], lane * 2048,
                               mask=lane < 15)
            return plsc.load_gather(lut_ref, [keyvec])

        slot2 = build_lut(lut2, b1)

        def slot_pass(p, shift, lut_prev_chain, T_in, slot_in):
            zero_hist()

            def sweep(i, _):
                for u in range(4):
                    k = keys_v[pl.ds(i * 64 + u * 16, 16)]
                    bits = lax.shift_right_logical(k, shift) & 127
                    base = lut_prev_chain(k)
                    idx = base + lane * 128 + bits
                    plsc.addupdate_scatter(hist, [idx], ones)
                return 0
            lax.fori_loop(0, CHUNK // 64, sweep, 0)
            lane_reduce_p2()
            publish_fetch(p)

            def bd(j, _):
                suffix_scan_chain(j * 8, 8)
                return 0
            lax.fori_loop(0, 16, bd, 0)

            slot_row = lax.shift_right_logical(slot_in, 11) * 128

            @plsc.parallel_loop(0, 128, unroll=8, carry=lane * 0)
            def bp(r, acc):
                sg = plsc.load_gather(S, [slot_row + r])
                return acc + (sg > T_in).astype(jnp.int32)
            Sg2 = plsc.load_gather(S, [slot_row + bp])
            T_out = T_in - Sg2
            return bp, T_out

        def chain2(k):
            b = lax.shift_right_logical(k, 21)
            return plsc.load_gather(lut2, [b])

        b2, T2 = slot_pass(1, 14, chain2, T1, slot2)

        key3 = lax.shift_right_logical(slot2, 11) * 128 + b2
        slot3 = build_lut(lut3, key3)

        def chain3(k):
            s2 = chain2(k)
            bits2 = lax.shift_right_logical(k, 14) & 127
            return plsc.load_gather(
                lut3, [lax.shift_right_logical(s2, 11) * 128 + bits2])

        b3, T3 = slot_pass(2, 7, chain3, T2, slot3)

        key4 = lax.shift_right_logical(slot3, 11) * 128 + b3
        slot4 = build_lut(lut4, key4)

        def chain4(k):
            s3 = chain3(k)
            bits3 = lax.shift_right_logical(k, 7) & 127
            return plsc.load_gather(
                lut4, [lax.shift_right_logical(s3, 11) * 128 + bits3])

        b4, _T4 = slot_pass(3, 0, chain4, T3, slot4)

        tvec = (b1 * (1 << 21)) + (b2 * (1 << 14)) + (b3 * (1 << 7)) + b4

        @pl.when(leader)
        def _out():
            tstage[...] = tvec
            pltpu.sync_copy(tstage, out_hbm.at[pl.ds(batch * 16, 16)])

    return sel(keys, jnp.asarray(_BK + [0x7FFFFFFF], dtype=jnp.int32))
